# baseline (speedup 1.0000x reference)
import functools

import jax
import jax.numpy as jnp
from jax import lax
from jax.experimental import pallas as pl
from jax.experimental.pallas import tpu as pltpu
from jax.experimental.pallas import tpu_sc as plsc

B = 4096
L = 50
V = 100000
D = 128
H = 128
C = 4

LP = 56
NLG = D // 16

BSC = 768
BTC = B - BSC

NW = 32
BPW = BSC // NW
CH = 2
NIDX = CH * LP
NCH = BPW // CH
NBUF = 4

TCBM = 256


def _pool_body(xc_hbm, table_hbm, out_hbm, idx_v, out_v, *bufs_and_sems):
    rows = bufs_and_sems[:NBUF]
    sems = bufs_and_sems[NBUF:]
    wid = lax.axis_index("s") * 2 + lax.axis_index("c")

    pltpu.sync_copy(xc_hbm.at[wid], idx_v)

    def start(c, b):
        pltpu.async_copy(table_hbm.at[idx_v.at[c]], rows[b], sems[b])

    for b in range(NBUF):
        start(b, b)

    def group(g, carry):
        for b in range(NBUF):
            c = g * NBUF + b
            pltpu.make_async_copy(
                table_hbm.at[idx_v.at[c]], rows[b], sems[b]
            ).wait()

            for r in range(CH):
                o = r * LP

                def tok(j, acc):
                    return tuple(
                        acc[k] + rows[b][o + j, pl.ds(16 * k, 16)]
                        for k in range(NLG)
                    )

                acc0 = tuple(rows[b][o, pl.ds(16 * k, 16)] for k in range(NLG))
                acc = lax.fori_loop(1, LP, tok, acc0, unroll=True)
                for k in range(NLG):
                    out_v[c * CH + r, pl.ds(16 * k, 16)] = acc[k]

            @pl.when(c + NBUF < NCH)
            def _():
                start(c + NBUF, b)
        return carry

    lax.fori_loop(0, NCH // NBUF, group, 0)
    pltpu.sync_copy(out_v, out_hbm.at[pl.ds(wid * BPW, BPW)])


@functools.cache
def _pool():
    return functools.partial(
        pl.kernel,
        mesh=plsc.VectorSubcoreMesh(core_axis_name="c", subcore_axis_name="s"),
        out_type=jax.ShapeDtypeStruct((BSC, D), jnp.float32),
        scratch_types=(
            [pltpu.VMEM((NCH, NIDX), jnp.int32),
             pltpu.VMEM((BPW, D), jnp.float32)]
            + [pltpu.VMEM((NIDX, D), jnp.float32) for _ in range(NBUF)]
            + [pltpu.SemaphoreType.DMA for _ in range(NBUF)]
        ),
    )(_pool_body)


def _tcpool_body(x_ref, table_ref, out_ref):
    def row(b, carry):
        accs = [table_ref[pl.ds(x_ref[b, l], 1), :] for l in range(4)]
        for l in range(4, L):
            accs[l % 4] = accs[l % 4] + table_ref[pl.ds(x_ref[b, l], 1), :]
        out_ref[pl.ds(b, 1), :] = (accs[0] + accs[1]) + (accs[2] + accs[3])
        return carry

    lax.fori_loop(0, TCBM, row, 0, unroll=16)


def _tcpool(xtc, table):
    return pl.pallas_call(
        _tcpool_body,
        grid=(BTC // TCBM,),
        in_specs=[
            pl.BlockSpec((TCBM, L), lambda i: (i, 0),
                         memory_space=pltpu.SMEM),
            pl.BlockSpec((V, D), lambda i: (0, 0)),
        ],
        out_specs=pl.BlockSpec((TCBM, D), lambda i: (i, 0)),
        out_shape=jax.ShapeDtypeStruct((BTC, D), jnp.float32),
    )(xtc, table)


def _mlp_body(s_ref, xl_ref, w1_ref, b1_ref, g_ref, bt_ref, mu_ref, var_ref,
              w2_ref, b2_ref, out_ref):
    s = s_ref[...]
    z = jnp.dot(s, w1_ref[...], preferred_element_type=jnp.float32)
    z = z / xl_ref[...] + b1_ref[...]
    a = g_ref[...] * lax.rsqrt(var_ref[...] + 1e-5)
    cshift = bt_ref[...] - mu_ref[...] * a
    h = jnp.maximum(z * a + cshift, 0.0)
    logits = jnp.dot(h, w2_ref[...], preferred_element_type=jnp.float32)
    logits = logits + b2_ref[...]
    col = lax.broadcasted_iota(jnp.int32, logits.shape, 1)
    valid = col < C
    masked = jnp.where(valid, logits, -jnp.inf)
    m = jnp.max(masked, axis=1, keepdims=True)
    e = jnp.where(valid, jnp.exp(logits - m), 0.0)
    lse = m + jnp.log(jnp.sum(e, axis=1, keepdims=True))
    out_ref[...] = logits - lse


def _mlp(s, xl, W1, b1, gamma, beta, mu, var, W2p, b2p):
    return pl.pallas_call(
        _mlp_body,
        out_shape=jax.ShapeDtypeStruct((B, D), jnp.float32),
    )(s, xl, W1, b1, gamma, beta, mu, var, W2p, b2p)


def kernel(x, x_len, table, W1, b1, gamma, beta, run_mean, run_var, W2, b2):
    xpad = jnp.zeros((B, LP), jnp.int32).at[:, :L].set(x.astype(jnp.int32))
    xc = xpad[:BSC].reshape(NW, NCH, NIDX)
    s_sc = _pool()(xc, table)
    s_tc = _tcpool(x[BSC:].astype(jnp.int32), table)
    s = jnp.concatenate([s_sc, s_tc], axis=0)
    xl = x_len.astype(jnp.float32).reshape(B, 1)
    W2p = jnp.zeros((H, D), jnp.float32).at[:, :C].set(W2)
    b2p = jnp.zeros((1, D), jnp.float32).at[0, :C].set(b2)
    out = _mlp(s, xl, W1, b1.reshape(1, H), gamma.reshape(1, H),
               beta.reshape(1, H), run_mean.reshape(1, H),
               run_var.reshape(1, H), W2p, b2p)
    return out[:, :C]


# speedup vs baseline: 1.0004x; 1.0004x over previous
import functools

import jax
import jax.numpy as jnp
from jax import lax
from jax.experimental import pallas as pl
from jax.experimental.pallas import tpu as pltpu
from jax.experimental.pallas import tpu_sc as plsc

B = 4096
L = 50
V = 100000
D = 128
H = 128
C = 4

LP = 56
NLG = D // 16

BSC = 768
BTC = B - BSC

NW = 32
BPW = BSC // NW
CH = 2
NIDX = CH * LP
NCH = BPW // CH
NBUF = 4

TCBM = 256


def _pool_body(xc_hbm, table_hbm, out_hbm, idx_v, out_v, *bufs_and_sems):
    rows = bufs_and_sems[:NBUF]
    sems = bufs_and_sems[NBUF:]
    wid = lax.axis_index("s") * 2 + lax.axis_index("c")

    pltpu.sync_copy(xc_hbm.at[wid], idx_v)

    def start(c, b):
        pltpu.async_copy(table_hbm.at[idx_v.at[c]], rows[b], sems[b])

    for b in range(NBUF):
        start(b, b)

    def group(g, carry):
        for b in range(NBUF):
            c = g * NBUF + b
            pltpu.make_async_copy(
                table_hbm.at[idx_v.at[c]], rows[b], sems[b]
            ).wait()

            for r in range(CH):
                o = r * LP

                def tok(j, acc):
                    return tuple(
                        acc[k] + rows[b][o + j, pl.ds(16 * k, 16)]
                        for k in range(NLG)
                    )

                acc0 = tuple(rows[b][o, pl.ds(16 * k, 16)] for k in range(NLG))
                acc = lax.fori_loop(1, LP, tok, acc0, unroll=True)
                for k in range(NLG):
                    out_v[c * CH + r, pl.ds(16 * k, 16)] = acc[k]

            @pl.when(c + NBUF < NCH)
            def _():
                start(c + NBUF, b)
        return carry

    lax.fori_loop(0, NCH // NBUF, group, 0)
    pltpu.sync_copy(out_v, out_hbm.at[pl.ds(wid * BPW, BPW)])


@functools.cache
def _pool():
    return functools.partial(
        pl.kernel,
        mesh=plsc.VectorSubcoreMesh(core_axis_name="c", subcore_axis_name="s"),
        out_type=jax.ShapeDtypeStruct((BSC, D), jnp.float32),
        scratch_types=(
            [pltpu.VMEM((NCH, NIDX), jnp.int32),
             pltpu.VMEM((BPW, D), jnp.float32)]
            + [pltpu.VMEM((NIDX, D), jnp.float32) for _ in range(NBUF)]
            + [pltpu.SemaphoreType.DMA for _ in range(NBUF)]
        ),
    )(_pool_body)


def _tcpool_body(x_ref, table_ref, out_ref):
    def row(b, carry):
        accs = [table_ref[pl.ds(x_ref[b, l], 1), :] for l in range(4)]
        for l in range(4, L):
            accs[l % 4] = accs[l % 4] + table_ref[pl.ds(x_ref[b, l], 1), :]
        out_ref[pl.ds(b, 1), :] = (accs[0] + accs[1]) + (accs[2] + accs[3])
        return carry

    lax.fori_loop(0, TCBM, row, 0, unroll=8)


def _tcpool(xtc, table):
    return pl.pallas_call(
        _tcpool_body,
        grid=(BTC // TCBM,),
        in_specs=[
            pl.BlockSpec((TCBM, L), lambda i: (i, 0),
                         memory_space=pltpu.SMEM),
            pl.BlockSpec((V, D), lambda i: (0, 0)),
        ],
        out_specs=pl.BlockSpec((TCBM, D), lambda i: (i, 0)),
        out_shape=jax.ShapeDtypeStruct((BTC, D), jnp.float32),
    )(xtc, table)


def _mlp_body(s_ref, xl_ref, w1_ref, b1_ref, g_ref, bt_ref, mu_ref, var_ref,
              w2_ref, b2_ref, out_ref):
    s = s_ref[...]
    z = jnp.dot(s, w1_ref[...], preferred_element_type=jnp.float32)
    z = z / xl_ref[...] + b1_ref[...]
    a = g_ref[...] * lax.rsqrt(var_ref[...] + 1e-5)
    cshift = bt_ref[...] - mu_ref[...] * a
    h = jnp.maximum(z * a + cshift, 0.0)
    logits = jnp.dot(h, w2_ref[...], preferred_element_type=jnp.float32)
    logits = logits + b2_ref[...]
    col = lax.broadcasted_iota(jnp.int32, logits.shape, 1)
    valid = col < C
    masked = jnp.where(valid, logits, -jnp.inf)
    m = jnp.max(masked, axis=1, keepdims=True)
    e = jnp.where(valid, jnp.exp(logits - m), 0.0)
    lse = m + jnp.log(jnp.sum(e, axis=1, keepdims=True))
    out_ref[...] = logits - lse


def _mlp(s, xl, W1, b1, gamma, beta, mu, var, W2p, b2p):
    return pl.pallas_call(
        _mlp_body,
        out_shape=jax.ShapeDtypeStruct((B, D), jnp.float32),
    )(s, xl, W1, b1, gamma, beta, mu, var, W2p, b2p)


def kernel(x, x_len, table, W1, b1, gamma, beta, run_mean, run_var, W2, b2):
    xpad = jnp.zeros((B, LP), jnp.int32).at[:, :L].set(x.astype(jnp.int32))
    xc = xpad[:BSC].reshape(NW, NCH, NIDX)
    s_sc = _pool()(xc, table)
    s_tc = _tcpool(x[BSC:].astype(jnp.int32), table)
    s = jnp.concatenate([s_sc, s_tc], axis=0)
    xl = x_len.astype(jnp.float32).reshape(B, 1)
    W2p = jnp.zeros((H, D), jnp.float32).at[:, :C].set(W2)
    b2p = jnp.zeros((1, D), jnp.float32).at[0, :C].set(b2)
    out = _mlp(s, xl, W1, b1.reshape(1, H), gamma.reshape(1, H),
               beta.reshape(1, H), run_mean.reshape(1, H),
               run_var.reshape(1, H), W2p, b2p)
    return out[:, :C]


# speedup vs baseline: 1.0016x; 1.0012x over previous
import functools

import jax
import jax.numpy as jnp
from jax import lax
from jax.experimental import pallas as pl
from jax.experimental.pallas import tpu as pltpu
from jax.experimental.pallas import tpu_sc as plsc

B = 4096
L = 50
V = 100000
D = 128
H = 128
C = 4

LP = 56
NLG = D // 16

BSC = 768
BTC = B - BSC

NW = 32
BPW = BSC // NW
CH = 2
NIDX = CH * LP
NCH = BPW // CH
NBUF = 4

TCBM = 256


def _pool_body(xc_hbm, table_hbm, out_hbm, idx_v, out_v, *bufs_and_sems):
    rows = bufs_and_sems[:NBUF]
    sems = bufs_and_sems[NBUF:]
    wid = lax.axis_index("s") * 2 + lax.axis_index("c")

    pltpu.sync_copy(xc_hbm.at[wid], idx_v)

    def start(c, b):
        pltpu.async_copy(table_hbm.at[idx_v.at[c]], rows[b], sems[b])

    for b in range(NBUF):
        start(b, b)

    def group(g, carry):
        for b in range(NBUF):
            c = g * NBUF + b
            pltpu.make_async_copy(
                table_hbm.at[idx_v.at[c]], rows[b], sems[b]
            ).wait()

            for r in range(CH):
                o = r * LP

                def tok(j, acc):
                    return tuple(
                        acc[k] + rows[b][o + j, pl.ds(16 * k, 16)]
                        for k in range(NLG)
                    )

                acc0 = tuple(rows[b][o, pl.ds(16 * k, 16)] for k in range(NLG))
                acc = lax.fori_loop(1, LP, tok, acc0, unroll=True)
                for k in range(NLG):
                    out_v[c * CH + r, pl.ds(16 * k, 16)] = acc[k]

            @pl.when(c + NBUF < NCH)
            def _():
                start(c + NBUF, b)
        return carry

    lax.fori_loop(0, NCH // NBUF, group, 0)
    pltpu.sync_copy(out_v, out_hbm.at[pl.ds(wid * BPW, BPW)])


@functools.cache
def _pool():
    return functools.partial(
        pl.kernel,
        mesh=plsc.VectorSubcoreMesh(core_axis_name="c", subcore_axis_name="s"),
        out_type=jax.ShapeDtypeStruct((BSC, D), jnp.float32),
        scratch_types=(
            [pltpu.VMEM((NCH, NIDX), jnp.int32),
             pltpu.VMEM((BPW, D), jnp.float32)]
            + [pltpu.VMEM((NIDX, D), jnp.float32) for _ in range(NBUF)]
            + [pltpu.SemaphoreType.DMA for _ in range(NBUF)]
        ),
    )(_pool_body)


def _tcpool_body(x_ref, table_ref, out_ref):
    def row(b, carry):
        accs = [table_ref[pl.ds(x_ref[b, l], 1), :] for l in range(4)]
        for l in range(4, L):
            accs[l % 4] = accs[l % 4] + table_ref[pl.ds(x_ref[b, l], 1), :]
        out_ref[pl.ds(b, 1), :] = (accs[0] + accs[1]) + (accs[2] + accs[3])
        return carry

    lax.fori_loop(0, TCBM, row, 0, unroll=8)


def _tcpool(xtc, table):
    return pl.pallas_call(
        _tcpool_body,
        grid=(BTC // TCBM,),
        in_specs=[
            pl.BlockSpec((TCBM, L), lambda i: (i, 0),
                         memory_space=pltpu.SMEM),
            pl.BlockSpec((V, D), lambda i: (0, 0)),
        ],
        out_specs=pl.BlockSpec((TCBM, D), lambda i: (i, 0)),
        out_shape=jax.ShapeDtypeStruct((BTC, D), jnp.float32),
    )(xtc, table)


def _mlp_body(s_ref, xl_ref, w1_ref, b1_ref, g_ref, bt_ref, mu_ref, var_ref,
              w2_ref, b2_ref, out_ref):
    s = s_ref[...]
    z = jnp.dot(s, w1_ref[...], preferred_element_type=jnp.float32)
    z = z / xl_ref[...] + b1_ref[...]
    a = g_ref[...] * lax.rsqrt(var_ref[...] + 1e-5)
    cshift = bt_ref[...] - mu_ref[...] * a
    h = jnp.maximum(z * a + cshift, 0.0)
    logits = jnp.dot(h, w2_ref[...], preferred_element_type=jnp.float32)
    logits = logits + b2_ref[...]
    col = lax.broadcasted_iota(jnp.int32, logits.shape, 1)
    valid = col < C
    masked = jnp.where(valid, logits, -jnp.inf)
    m = jnp.max(masked, axis=1, keepdims=True)
    e = jnp.where(valid, jnp.exp(logits - m), 0.0)
    lse = m + jnp.log(jnp.sum(e, axis=1, keepdims=True))
    out_ref[...] = lax.slice(logits - lse, (0, 0), (logits.shape[0], C))


def _mlp(s, xl, W1, b1, gamma, beta, mu, var, W2p, b2p):
    return pl.pallas_call(
        _mlp_body,
        out_shape=jax.ShapeDtypeStruct((B, C), jnp.float32),
    )(s, xl, W1, b1, gamma, beta, mu, var, W2p, b2p)


def kernel(x, x_len, table, W1, b1, gamma, beta, run_mean, run_var, W2, b2):
    xi = x.astype(jnp.int32)
    xpad = jnp.zeros((BSC, LP), jnp.int32).at[:, :L].set(xi[:BSC])
    xc = xpad.reshape(NW, NCH, NIDX)
    s_sc = _pool()(xc, table)
    s_tc = _tcpool(xi[BSC:], table)
    s = jnp.concatenate([s_sc, s_tc], axis=0)
    xl = x_len.astype(jnp.float32).reshape(B, 1)
    W2p = jnp.zeros((H, D), jnp.float32).at[:, :C].set(W2)
    b2p = jnp.zeros((1, D), jnp.float32).at[0, :C].set(b2)
    return _mlp(s, xl, W1, b1.reshape(1, H), gamma.reshape(1, H),
                beta.reshape(1, H), run_mean.reshape(1, H),
                run_var.reshape(1, H), W2p, b2p)


# speedup vs baseline: 1.2256x; 1.2236x over previous
import functools

import jax
import jax.numpy as jnp
from jax import lax
from jax.experimental import pallas as pl
from jax.experimental.pallas import tpu as pltpu
from jax.experimental.pallas import tpu_sc as plsc

B = 4096
L = 50
V = 100000
D = 128
H = 128
C = 4

NLG = D // 16

BSC = 768
BTC = B - BSC

NW = 32
BPW = BSC // NW
NCH = BPW
NBUF = 4

TCBM = 256


def _pool_body(x3_hbm, table_hbm, out_hbm, idx_v, out_v, *bufs_and_sems):
    rows = bufs_and_sems[:NBUF]
    sems = bufs_and_sems[NBUF:]
    wid = lax.axis_index("s") * 2 + lax.axis_index("c")

    pltpu.sync_copy(x3_hbm.at[wid], idx_v)

    def start(c, b):
        pltpu.async_copy(table_hbm.at[idx_v.at[c]], rows[b], sems[b])

    for b in range(NBUF):
        start(b, b)

    def group(g, carry):
        for b in range(NBUF):
            c = g * NBUF + b
            pltpu.make_async_copy(
                table_hbm.at[idx_v.at[c]], rows[b], sems[b]
            ).wait()

            def tok(j, acc):
                return tuple(
                    acc[k] + rows[b][j, pl.ds(16 * k, 16)]
                    for k in range(NLG)
                )

            acc0 = tuple(rows[b][0, pl.ds(16 * k, 16)]
                         for k in range(NLG))
            acc = lax.fori_loop(1, L, tok, acc0, unroll=True)
            for k in range(NLG):
                out_v[c, pl.ds(16 * k, 16)] = acc[k]

            @pl.when(c + NBUF < NCH)
            def _():
                start(c + NBUF, b)
        return carry

    lax.fori_loop(0, NCH // NBUF, group, 0)
    pltpu.sync_copy(out_v, out_hbm.at[pl.ds(wid * BPW, BPW)])


@functools.cache
def _pool():
    return functools.partial(
        pl.kernel,
        mesh=plsc.VectorSubcoreMesh(core_axis_name="c", subcore_axis_name="s"),
        out_type=jax.ShapeDtypeStruct((BSC, D), jnp.float32),
        scratch_types=(
            [pltpu.VMEM((BPW, L), jnp.int32),
             pltpu.VMEM((BPW, D), jnp.float32)]
            + [pltpu.VMEM((L, D), jnp.float32) for _ in range(NBUF)]
            + [pltpu.SemaphoreType.DMA for _ in range(NBUF)]
        ),
    )(_pool_body)


def _head(s, xl, w1, b1, g, bt, mu, var, w2, b2):
    z = jnp.dot(s, w1, preferred_element_type=jnp.float32)
    z = z / xl + b1
    a = g * lax.rsqrt(var + 1e-5)
    cshift = bt - mu * a
    h = jnp.maximum(z * a + cshift, 0.0)
    logits = jnp.dot(h, w2, preferred_element_type=jnp.float32) + b2
    col = lax.broadcasted_iota(jnp.int32, logits.shape, 1)
    valid = col < C
    masked = jnp.where(valid, logits, -jnp.inf)
    m = jnp.max(masked, axis=1, keepdims=True)
    e = jnp.where(valid, jnp.exp(logits - m), 0.0)
    lse = m + jnp.log(jnp.sum(e, axis=1, keepdims=True))
    return lax.slice(logits - lse, (0, 0), (logits.shape[0], C))


def _tcpool_body(x_ref, table_ref, xl_ref, w1_ref, b1_ref, g_ref, bt_ref,
                 mu_ref, var_ref, w2_ref, b2_ref, out_ref, s_ref):
    def row(b, carry):
        accs = [table_ref[pl.ds(x_ref[b, l], 1), :] for l in range(4)]
        for l in range(4, L):
            accs[l % 4] = accs[l % 4] + table_ref[pl.ds(x_ref[b, l], 1), :]
        s_ref[pl.ds(b, 1), :] = (accs[0] + accs[1]) + (accs[2] + accs[3])
        return carry

    lax.fori_loop(0, TCBM, row, 0, unroll=8)
    out_ref[...] = _head(s_ref[...], xl_ref[...], w1_ref[...], b1_ref[...],
                         g_ref[...], bt_ref[...], mu_ref[...], var_ref[...],
                         w2_ref[...], b2_ref[...])


def _tcpool(xtc, table, xltc, W1, b1, gamma, beta, mu, var, W2p, b2p):
    const = lambda i: (0, 0)
    return pl.pallas_call(
        _tcpool_body,
        grid=(BTC // TCBM,),
        in_specs=[
            pl.BlockSpec((TCBM, L), lambda i: (i, 0),
                         memory_space=pltpu.SMEM),
            pl.BlockSpec((V, D), const),
            pl.BlockSpec((TCBM, 1), lambda i: (i, 0)),
            pl.BlockSpec((H, H), const),
            pl.BlockSpec((1, H), const),
            pl.BlockSpec((1, H), const),
            pl.BlockSpec((1, H), const),
            pl.BlockSpec((1, H), const),
            pl.BlockSpec((1, H), const),
            pl.BlockSpec((H, D), const),
            pl.BlockSpec((1, D), const),
        ],
        out_specs=pl.BlockSpec((TCBM, C), lambda i: (i, 0)),
        out_shape=jax.ShapeDtypeStruct((BTC, C), jnp.float32),
        scratch_shapes=[pltpu.VMEM((TCBM, D), jnp.float32)],
    )(xtc, table, xltc, W1, b1, gamma, beta, mu, var, W2p, b2p)


def _mlp_body(s_ref, xl_ref, w1_ref, b1_ref, g_ref, bt_ref, mu_ref, var_ref,
              w2_ref, b2_ref, out_ref):
    out_ref[...] = _head(s_ref[...], xl_ref[...], w1_ref[...], b1_ref[...],
                         g_ref[...], bt_ref[...], mu_ref[...], var_ref[...],
                         w2_ref[...], b2_ref[...])


def _mlp(s, xl, W1, b1, gamma, beta, mu, var, W2p, b2p):
    return pl.pallas_call(
        _mlp_body,
        out_shape=jax.ShapeDtypeStruct((BSC, C), jnp.float32),
    )(s, xl, W1, b1, gamma, beta, mu, var, W2p, b2p)


def kernel(x, x_len, table, W1, b1, gamma, beta, run_mean, run_var, W2, b2):
    xi = x.astype(jnp.int32)
    x3 = xi[:BSC].reshape(NW, BPW, L)
    s_sc = _pool()(x3, table)
    xl = x_len.astype(jnp.float32).reshape(B, 1)
    W2p = jnp.zeros((H, D), jnp.float32).at[:, :C].set(W2)
    b2p = jnp.zeros((1, D), jnp.float32).at[0, :C].set(b2)
    b1r = b1.reshape(1, H)
    gr = gamma.reshape(1, H)
    btr = beta.reshape(1, H)
    mur = run_mean.reshape(1, H)
    varr = run_var.reshape(1, H)
    out_tc = _tcpool(xi[BSC:], table, xl[BSC:], W1, b1r, gr, btr, mur, varr,
                     W2p, b2p)
    out_sc = _mlp(s_sc, xl[:BSC], W1, b1r, gr, btr, mur, varr, W2p, b2p)
    return jnp.concatenate([out_sc, out_tc], axis=0)


# speedup vs baseline: 1.9756x; 1.6119x over previous
import functools

import jax
import jax.numpy as jnp
from jax import lax
from jax.experimental import pallas as pl
from jax.experimental.pallas import tpu as pltpu
from jax.experimental.pallas import tpu_sc as plsc

B = 4096
L = 50
V = 100000
D = 128
H = 128
C = 4

NLG = D // 16

BSC = 2560
BTC = B - BSC

NW = 32
BPW = BSC // NW
NCH = BPW
NBUF = 4

TCBM = 256


def _pool_body(x3_hbm, table_hbm, out_hbm, idx_v, out_v, *bufs_and_sems):
    rows = bufs_and_sems[:NBUF]
    sems = bufs_and_sems[NBUF:]
    wid = lax.axis_index("s") * 2 + lax.axis_index("c")

    pltpu.sync_copy(x3_hbm.at[wid], idx_v)

    def start(c, b):
        pltpu.async_copy(table_hbm.at[idx_v.at[c]], rows[b], sems[b])

    for b in range(NBUF):
        start(b, b)

    def group(g, carry):
        for b in range(NBUF):
            c = g * NBUF + b
            pltpu.make_async_copy(
                table_hbm.at[idx_v.at[c]], rows[b], sems[b]
            ).wait()

            def tok(j, acc):
                return tuple(
                    acc[k] + rows[b][j, pl.ds(16 * k, 16)]
                    for k in range(NLG)
                )

            acc0 = tuple(rows[b][0, pl.ds(16 * k, 16)]
                         for k in range(NLG))
            acc = lax.fori_loop(1, L, tok, acc0, unroll=True)
            for k in range(NLG):
                out_v[c, pl.ds(16 * k, 16)] = acc[k]

            @pl.when(c + NBUF < NCH)
            def _():
                start(c + NBUF, b)
        return carry

    lax.fori_loop(0, NCH // NBUF, group, 0)
    pltpu.sync_copy(out_v, out_hbm.at[pl.ds(wid * BPW, BPW)])


@functools.cache
def _pool():
    return functools.partial(
        pl.kernel,
        mesh=plsc.VectorSubcoreMesh(core_axis_name="c", subcore_axis_name="s"),
        out_type=jax.ShapeDtypeStruct((BSC, D), jnp.float32),
        scratch_types=(
            [pltpu.VMEM((BPW, L), jnp.int32),
             pltpu.VMEM((BPW, D), jnp.float32)]
            + [pltpu.VMEM((L, D), jnp.float32) for _ in range(NBUF)]
            + [pltpu.SemaphoreType.DMA for _ in range(NBUF)]
        ),
    )(_pool_body)


def _head(s, xl, w1, b1, g, bt, mu, var, w2, b2):
    z = jnp.dot(s, w1, preferred_element_type=jnp.float32)
    z = z / xl + b1
    a = g * lax.rsqrt(var + 1e-5)
    cshift = bt - mu * a
    h = jnp.maximum(z * a + cshift, 0.0)
    logits = jnp.dot(h, w2, preferred_element_type=jnp.float32) + b2
    col = lax.broadcasted_iota(jnp.int32, logits.shape, 1)
    valid = col < C
    masked = jnp.where(valid, logits, -jnp.inf)
    m = jnp.max(masked, axis=1, keepdims=True)
    e = jnp.where(valid, jnp.exp(logits - m), 0.0)
    lse = m + jnp.log(jnp.sum(e, axis=1, keepdims=True))
    return lax.slice(logits - lse, (0, 0), (logits.shape[0], C))


def _tcpool_body(x_ref, table_ref, xl_ref, w1_ref, b1_ref, g_ref, bt_ref,
                 mu_ref, var_ref, w2_ref, b2_ref, out_ref, s_ref):
    def row(b, carry):
        accs = [table_ref[pl.ds(x_ref[b, l], 1), :] for l in range(4)]
        for l in range(4, L):
            accs[l % 4] = accs[l % 4] + table_ref[pl.ds(x_ref[b, l], 1), :]
        s_ref[pl.ds(b, 1), :] = (accs[0] + accs[1]) + (accs[2] + accs[3])
        return carry

    lax.fori_loop(0, TCBM, row, 0, unroll=8)
    out_ref[...] = _head(s_ref[...], xl_ref[...], w1_ref[...], b1_ref[...],
                         g_ref[...], bt_ref[...], mu_ref[...], var_ref[...],
                         w2_ref[...], b2_ref[...])


def _tcpool(xtc, table, xltc, W1, b1, gamma, beta, mu, var, W2p, b2p):
    const = lambda i: (0, 0)
    return pl.pallas_call(
        _tcpool_body,
        grid=(BTC // TCBM,),
        in_specs=[
            pl.BlockSpec((TCBM, L), lambda i: (i, 0),
                         memory_space=pltpu.SMEM),
            pl.BlockSpec((V, D), const),
            pl.BlockSpec((TCBM, 1), lambda i: (i, 0)),
            pl.BlockSpec((H, H), const),
            pl.BlockSpec((1, H), const),
            pl.BlockSpec((1, H), const),
            pl.BlockSpec((1, H), const),
            pl.BlockSpec((1, H), const),
            pl.BlockSpec((1, H), const),
            pl.BlockSpec((H, D), const),
            pl.BlockSpec((1, D), const),
        ],
        out_specs=pl.BlockSpec((TCBM, C), lambda i: (i, 0)),
        out_shape=jax.ShapeDtypeStruct((BTC, C), jnp.float32),
        scratch_shapes=[pltpu.VMEM((TCBM, D), jnp.float32)],
    )(xtc, table, xltc, W1, b1, gamma, beta, mu, var, W2p, b2p)


def _mlp_body(s_ref, xl_ref, w1_ref, b1_ref, g_ref, bt_ref, mu_ref, var_ref,
              w2_ref, b2_ref, out_ref):
    out_ref[...] = _head(s_ref[...], xl_ref[...], w1_ref[...], b1_ref[...],
                         g_ref[...], bt_ref[...], mu_ref[...], var_ref[...],
                         w2_ref[...], b2_ref[...])


def _mlp(s, xl, W1, b1, gamma, beta, mu, var, W2p, b2p):
    return pl.pallas_call(
        _mlp_body,
        out_shape=jax.ShapeDtypeStruct((BSC, C), jnp.float32),
    )(s, xl, W1, b1, gamma, beta, mu, var, W2p, b2p)


def kernel(x, x_len, table, W1, b1, gamma, beta, run_mean, run_var, W2, b2):
    xi = x.astype(jnp.int32)
    x3 = xi[:BSC].reshape(NW, BPW, L)
    s_sc = _pool()(x3, table)
    xl = x_len.astype(jnp.float32).reshape(B, 1)
    W2p = jnp.zeros((H, D), jnp.float32).at[:, :C].set(W2)
    b2p = jnp.zeros((1, D), jnp.float32).at[0, :C].set(b2)
    b1r = b1.reshape(1, H)
    gr = gamma.reshape(1, H)
    btr = beta.reshape(1, H)
    mur = run_mean.reshape(1, H)
    varr = run_var.reshape(1, H)
    out_tc = _tcpool(xi[BSC:], table, xl[BSC:], W1, b1r, gr, btr, mur, varr,
                     W2p, b2p)
    out_sc = _mlp(s_sc, xl[:BSC], W1, b1r, gr, btr, mur, varr, W2p, b2p)
    return jnp.concatenate([out_sc, out_tc], axis=0)


# speedup vs baseline: 1.9791x; 1.0018x over previous
import functools

import jax
import jax.numpy as jnp
from jax import lax
from jax.experimental import pallas as pl
from jax.experimental.pallas import tpu as pltpu
from jax.experimental.pallas import tpu_sc as plsc

B = 4096
L = 50
V = 100000
D = 128
H = 128
C = 4

NLG = D // 16

BSC = 2816
BTC = B - BSC

NW = 32
BPW = BSC // NW
NCH = BPW
NBUF = 4

TCBM = 256


def _pool_body(x3_hbm, table_hbm, out_hbm, idx_v, out_v, *bufs_and_sems):
    rows = bufs_and_sems[:NBUF]
    sems = bufs_and_sems[NBUF:]
    wid = lax.axis_index("s") * 2 + lax.axis_index("c")

    pltpu.sync_copy(x3_hbm.at[wid], idx_v)

    def start(c, b):
        pltpu.async_copy(table_hbm.at[idx_v.at[c]], rows[b], sems[b])

    for b in range(NBUF):
        start(b, b)

    def group(g, carry):
        for b in range(NBUF):
            c = g * NBUF + b
            pltpu.make_async_copy(
                table_hbm.at[idx_v.at[c]], rows[b], sems[b]
            ).wait()

            def tok(j, acc):
                return tuple(
                    acc[k] + rows[b][j, pl.ds(16 * k, 16)]
                    for k in range(NLG)
                )

            acc0 = tuple(rows[b][0, pl.ds(16 * k, 16)]
                         for k in range(NLG))
            acc = lax.fori_loop(1, L, tok, acc0, unroll=True)
            for k in range(NLG):
                out_v[c, pl.ds(16 * k, 16)] = acc[k]

            @pl.when(c + NBUF < NCH)
            def _():
                start(c + NBUF, b)
        return carry

    lax.fori_loop(0, NCH // NBUF, group, 0)
    pltpu.sync_copy(out_v, out_hbm.at[pl.ds(wid * BPW, BPW)])


@functools.cache
def _pool():
    return functools.partial(
        pl.kernel,
        mesh=plsc.VectorSubcoreMesh(core_axis_name="c", subcore_axis_name="s"),
        out_type=jax.ShapeDtypeStruct((BSC, D), jnp.float32),
        scratch_types=(
            [pltpu.VMEM((BPW, L), jnp.int32),
             pltpu.VMEM((BPW, D), jnp.float32)]
            + [pltpu.VMEM((L, D), jnp.float32) for _ in range(NBUF)]
            + [pltpu.SemaphoreType.DMA for _ in range(NBUF)]
        ),
    )(_pool_body)


def _head(s, xl, w1, b1, g, bt, mu, var, w2, b2):
    z = jnp.dot(s, w1, preferred_element_type=jnp.float32)
    z = z / xl + b1
    a = g * lax.rsqrt(var + 1e-5)
    cshift = bt - mu * a
    h = jnp.maximum(z * a + cshift, 0.0)
    logits = jnp.dot(h, w2, preferred_element_type=jnp.float32) + b2
    col = lax.broadcasted_iota(jnp.int32, logits.shape, 1)
    valid = col < C
    masked = jnp.where(valid, logits, -jnp.inf)
    m = jnp.max(masked, axis=1, keepdims=True)
    e = jnp.where(valid, jnp.exp(logits - m), 0.0)
    lse = m + jnp.log(jnp.sum(e, axis=1, keepdims=True))
    return lax.slice(logits - lse, (0, 0), (logits.shape[0], C))


def _tcpool_body(x_ref, table_ref, xl_ref, w1_ref, b1_ref, g_ref, bt_ref,
                 mu_ref, var_ref, w2_ref, b2_ref, out_ref, s_ref):
    def row(b, carry):
        accs = [table_ref[pl.ds(x_ref[b, l], 1), :] for l in range(4)]
        for l in range(4, L):
            accs[l % 4] = accs[l % 4] + table_ref[pl.ds(x_ref[b, l], 1), :]
        s_ref[pl.ds(b, 1), :] = (accs[0] + accs[1]) + (accs[2] + accs[3])
        return carry

    lax.fori_loop(0, TCBM, row, 0, unroll=8)
    out_ref[...] = _head(s_ref[...], xl_ref[...], w1_ref[...], b1_ref[...],
                         g_ref[...], bt_ref[...], mu_ref[...], var_ref[...],
                         w2_ref[...], b2_ref[...])


def _tcpool(xtc, table, xltc, W1, b1, gamma, beta, mu, var, W2p, b2p):
    const = lambda i: (0, 0)
    return pl.pallas_call(
        _tcpool_body,
        grid=(BTC // TCBM,),
        in_specs=[
            pl.BlockSpec((TCBM, L), lambda i: (i, 0),
                         memory_space=pltpu.SMEM),
            pl.BlockSpec((V, D), const),
            pl.BlockSpec((TCBM, 1), lambda i: (i, 0)),
            pl.BlockSpec((H, H), const),
            pl.BlockSpec((1, H), const),
            pl.BlockSpec((1, H), const),
            pl.BlockSpec((1, H), const),
            pl.BlockSpec((1, H), const),
            pl.BlockSpec((1, H), const),
            pl.BlockSpec((H, D), const),
            pl.BlockSpec((1, D), const),
        ],
        out_specs=pl.BlockSpec((TCBM, C), lambda i: (i, 0)),
        out_shape=jax.ShapeDtypeStruct((BTC, C), jnp.float32),
        scratch_shapes=[pltpu.VMEM((TCBM, D), jnp.float32)],
    )(xtc, table, xltc, W1, b1, gamma, beta, mu, var, W2p, b2p)


def _mlp_body(s_ref, xl_ref, w1_ref, b1_ref, g_ref, bt_ref, mu_ref, var_ref,
              w2_ref, b2_ref, out_ref):
    out_ref[...] = _head(s_ref[...], xl_ref[...], w1_ref[...], b1_ref[...],
                         g_ref[...], bt_ref[...], mu_ref[...], var_ref[...],
                         w2_ref[...], b2_ref[...])


def _mlp(s, xl, W1, b1, gamma, beta, mu, var, W2p, b2p):
    return pl.pallas_call(
        _mlp_body,
        out_shape=jax.ShapeDtypeStruct((BSC, C), jnp.float32),
    )(s, xl, W1, b1, gamma, beta, mu, var, W2p, b2p)


def kernel(x, x_len, table, W1, b1, gamma, beta, run_mean, run_var, W2, b2):
    xi = x.astype(jnp.int32)
    x3 = xi[:BSC].reshape(NW, BPW, L)
    s_sc = _pool()(x3, table)
    xl = x_len.astype(jnp.float32).reshape(B, 1)
    W2p = jnp.zeros((H, D), jnp.float32).at[:, :C].set(W2)
    b2p = jnp.zeros((1, D), jnp.float32).at[0, :C].set(b2)
    b1r = b1.reshape(1, H)
    gr = gamma.reshape(1, H)
    btr = beta.reshape(1, H)
    mur = run_mean.reshape(1, H)
    varr = run_var.reshape(1, H)
    out_tc = _tcpool(xi[BSC:], table, xl[BSC:], W1, b1r, gr, btr, mur, varr,
                     W2p, b2p)
    out_sc = _mlp(s_sc, xl[:BSC], W1, b1r, gr, btr, mur, varr, W2p, b2p)
    return jnp.concatenate([out_sc, out_tc], axis=0)
